# baseline (speedup 1.0000x reference)
import functools

import jax
import jax.numpy as jnp
from jax import lax
from jax.experimental import pallas as pl
from jax.experimental.pallas import tpu as pltpu
from jax.experimental.pallas import tpu_sc as plsc

VOCAB = 100000
EMBED = 64
N_ROWS = 16384
N_COLS = 50

NUM_CORES = 2
NUM_SUBCORES = 16
NW = NUM_CORES * NUM_SUBCORES
R_PER_W = N_ROWS // NW
BT_PER_W = R_PER_W // 128
NUNITS = N_COLS * BT_PER_W


def _make_gather():
    mesh = plsc.VectorSubcoreMesh(core_axis_name="c", subcore_axis_name="s")

    @functools.partial(
        pl.kernel,
        mesh=mesh,
        out_type=jax.ShapeDtypeStruct((N_COLS, 8, 128, 8, 128), jnp.float32),
        scratch_types=[
            pltpu.VMEM((R_PER_W, N_COLS), jnp.int32),
            pltpu.VMEM((N_COLS, BT_PER_W, 128), jnp.int32),
            pltpu.VMEM((128, EMBED), jnp.float32),
            pltpu.VMEM((128, EMBED), jnp.float32),
            pltpu.VMEM((8, 8, 128), jnp.float32),
            pltpu.VMEM((8, 8, 128), jnp.float32),
            pltpu.SemaphoreType.DMA,
            pltpu.SemaphoreType.DMA,
            pltpu.SemaphoreType.DMA,
            pltpu.SemaphoreType.DMA,
        ],
        compiler_params=pltpu.CompilerParams(
            use_tc_tiling_on_sc=False, needs_layout_passes=False),
    )
    def gather_kernel(idx_hbm, table_hbm, out_hbm, idx_v, idx_t,
                      rows0, rows1, tr0, tr1, g0, g1, s0, s1):
        wid = lax.axis_index("s") * NUM_CORES + lax.axis_index("c")
        base_row = wid * R_PER_W
        pltpu.sync_copy(idx_hbm.at[pl.ds(base_row, R_PER_W)], idx_v)

        iota16 = lax.iota(jnp.int32, 16)

        def build_col(s, carry):
            sv = lax.broadcast(s, (16,))
            for bloc in range(BT_PER_W):
                for bc in range(8):
                    rowv = iota16 + (bloc * 128 + bc * 16)
                    v = plsc.load_gather(idx_v, [rowv, sv])
                    idx_t[s, bloc, pl.ds(bc * 16, 16)] = v
            return carry

        lax.fori_loop(0, N_COLS, build_col, 0)

        def fg(u, rows, gsem):
            s = u // BT_PER_W
            bloc = lax.rem(u, BT_PER_W)
            pltpu.async_copy(table_hbm.at[idx_t.at[s, bloc]], rows, gsem)

        def dg(rows, gsem):
            pltpu.make_async_copy(
                table_hbm.at[pl.ds(0, 128)], rows, gsem).wait()

        def transpose(rows, tr):
            for et in range(8):
                for ei in range(8):
                    ev = lax.broadcast(jnp.int32(et * 8 + ei), (16,))
                    for bc in range(8):
                        bv = iota16 + bc * 16
                        v = plsc.load_gather(rows, [bv, ev])
                        tr[et, ei, pl.ds(bc * 16, 16)] = v

        def fs(u, tr, ssem):
            s = u // BT_PER_W
            btg = wid * BT_PER_W + lax.rem(u, BT_PER_W)
            for et in range(8):
                pltpu.async_copy(tr.at[et], out_hbm.at[s, et, btg], ssem)

        def ds(tr, ssem):
            for et in range(8):
                pltpu.make_async_copy(
                    tr.at[et], out_hbm.at[0, 0, 0], ssem).wait()

        def unit(u, rows, tr, gsem, ssem, first):
            dg(rows, gsem)
            if not first:
                ds(tr, ssem)
            transpose(rows, tr)

            @pl.when(u + 2 < NUNITS)
            def _():
                fg(u + 2, rows, gsem)

            fs(u, tr, ssem)

        fg(0, rows0, g0)
        fg(1, rows1, g1)
        unit(jnp.int32(0), rows0, tr0, g0, s0, first=True)
        unit(jnp.int32(1), rows1, tr1, g1, s1, first=True)

        def body(p, carry):
            unit(2 * p, rows0, tr0, g0, s0, first=False)
            unit(2 * p + 1, rows1, tr1, g1, s1, first=False)
            return carry

        lax.fori_loop(1, NUNITS // 2, body, 0)

        ds(tr0, s0)
        ds(tr1, s1)

    return gather_kernel


_gather = _make_gather()


def kernel(indices, table):
    p = _gather(indices.astype(jnp.int32), table)
    return jnp.transpose(p, (2, 4, 0, 1, 3)).reshape(N_ROWS, N_COLS, EMBED)


# speedup vs baseline: 2.0308x; 2.0308x over previous
import functools

import jax
import jax.numpy as jnp
from jax import lax
from jax.experimental import pallas as pl
from jax.experimental.pallas import tpu as pltpu
from jax.experimental.pallas import tpu_sc as plsc

VOCAB = 100000
EMBED = 64
N_ROWS = 16384
N_COLS = 50

NUM_CORES = 2
NUM_SUBCORES = 16
NW = NUM_CORES * NUM_SUBCORES
R_PER_W = N_ROWS // NW
BT_PER_W = R_PER_W // 128
NUNITS = N_COLS * BT_PER_W
NSLOTS = 4


def _make_gather():
    mesh = plsc.VectorSubcoreMesh(core_axis_name="c", subcore_axis_name="s")

    @functools.partial(
        pl.kernel,
        mesh=mesh,
        out_type=jax.ShapeDtypeStruct((N_COLS, N_ROWS, EMBED), jnp.float32),
        scratch_types=[
            pltpu.VMEM((R_PER_W, N_COLS), jnp.int32),
            pltpu.VMEM((N_COLS, BT_PER_W, 128), jnp.int32),
            pltpu.VMEM((NSLOTS, 128, EMBED), jnp.float32),
            pltpu.SemaphoreType.DMA,
            pltpu.SemaphoreType.DMA,
        ],
        compiler_params=pltpu.CompilerParams(
            use_tc_tiling_on_sc=False, needs_layout_passes=False),
    )
    def gather_kernel(idx_hbm, table_hbm, out_hbm, idx_v, idx_t, rows,
                      gsem, ssem):
        wid = lax.axis_index("s") * NUM_CORES + lax.axis_index("c")
        base_row = wid * R_PER_W
        pltpu.sync_copy(idx_hbm.at[pl.ds(base_row, R_PER_W)], idx_v)

        iota16 = lax.iota(jnp.int32, 16)

        def build_col(s, carry):
            sv = lax.broadcast(s, (16,))
            for bloc in range(BT_PER_W):
                for bc in range(8):
                    rowv = iota16 + (bloc * 128 + bc * 16)
                    v = plsc.load_gather(idx_v, [rowv, sv])
                    idx_t[s, bloc, pl.ds(bc * 16, 16)] = v
            return carry

        lax.fori_loop(0, N_COLS, build_col, 0)

        def fg(u):
            s = u // BT_PER_W
            bloc = lax.rem(u, BT_PER_W)
            slot = lax.rem(u, NSLOTS)
            pltpu.async_copy(
                table_hbm.at[idx_t.at[s, bloc]], rows.at[slot], gsem)

        def fs(u):
            s = u // BT_PER_W
            bloc = lax.rem(u, BT_PER_W)
            slot = lax.rem(u, NSLOTS)
            b0 = base_row + bloc * 128
            pltpu.async_copy(
                rows.at[slot], out_hbm.at[s, pl.ds(b0, 128)], ssem)

        def dg():
            pltpu.make_async_copy(
                table_hbm.at[pl.ds(0, 128)], rows.at[0], gsem).wait()

        def ds():
            pltpu.make_async_copy(
                rows.at[0], out_hbm.at[0, pl.ds(0, 128)], ssem).wait()

        fg(jnp.int32(0))
        fg(jnp.int32(1))

        def body(u, carry):
            dg()

            @pl.when(u >= 2)
            def _():
                ds()

            @pl.when(u + 2 < NUNITS)
            def _():
                fg(u + 2)

            fs(u)
            return carry

        lax.fori_loop(0, NUNITS, body, 0)
        ds()
        ds()

    return gather_kernel


_gather = _make_gather()


def kernel(indices, table):
    l = _gather(indices.astype(jnp.int32), table)
    return jnp.transpose(l, (1, 0, 2))


# speedup vs baseline: 3.2670x; 1.6087x over previous
import functools

import jax
import jax.numpy as jnp
import numpy as np
from jax import lax
from jax.experimental import pallas as pl
from jax.experimental.pallas import tpu as pltpu
from jax.experimental.pallas import tpu_sc as plsc

VOCAB = 100000
EMBED = 64
N_ROWS = 16384
N_COLS = 50

NUM_CORES = 2
NUM_SUBCORES = 16
NW = NUM_CORES * NUM_SUBCORES
R_PER_W = N_ROWS // NW
BT_PER_W = R_PER_W // 128
NUNITS = N_COLS * BT_PER_W
NSLOTS = 4


def _make_gather():
    mesh = plsc.VectorSubcoreMesh(core_axis_name="c", subcore_axis_name="s")

    @functools.partial(
        pl.kernel,
        mesh=mesh,
        out_type=jax.ShapeDtypeStruct((N_COLS, N_ROWS, EMBED), jnp.float32),
        scratch_types=[
            pltpu.VMEM((R_PER_W, N_COLS), jnp.int32),
            pltpu.VMEM((N_COLS, BT_PER_W, 128), jnp.int32),
            pltpu.VMEM((NSLOTS, 128, EMBED), jnp.float32),
            pltpu.SemaphoreType.DMA,
            pltpu.SemaphoreType.DMA,
        ],
        compiler_params=pltpu.CompilerParams(
            use_tc_tiling_on_sc=False, needs_layout_passes=False),
    )
    def gather_kernel(idx_hbm, table_hbm, out_hbm, idx_v, idx_t, rows,
                      gsem, ssem):
        wid = lax.axis_index("s") * NUM_CORES + lax.axis_index("c")
        base_row = wid * R_PER_W
        pltpu.sync_copy(idx_hbm.at[pl.ds(base_row, R_PER_W)], idx_v)

        iota16 = lax.iota(jnp.int32, 16)

        def build_col(s, carry):
            sv = lax.broadcast(s, (16,))
            for bloc in range(BT_PER_W):
                for bc in range(8):
                    rowv = iota16 + (bloc * 128 + bc * 16)
                    v = plsc.load_gather(idx_v, [rowv, sv])
                    idx_t[s, bloc, pl.ds(bc * 16, 16)] = v
            return carry

        lax.fori_loop(0, N_COLS, build_col, 0)

        def fg(u):
            s = u // BT_PER_W
            bloc = lax.rem(u, BT_PER_W)
            slot = lax.rem(u, NSLOTS)
            pltpu.async_copy(
                table_hbm.at[idx_t.at[s, bloc]], rows.at[slot], gsem)

        def fs(u):
            s = u // BT_PER_W
            bloc = lax.rem(u, BT_PER_W)
            slot = lax.rem(u, NSLOTS)
            b0 = base_row + bloc * 128
            pltpu.async_copy(
                rows.at[slot], out_hbm.at[s, pl.ds(b0, 128)], ssem)

        def dg():
            pltpu.make_async_copy(
                table_hbm.at[pl.ds(0, 128)], rows.at[0], gsem).wait()

        def ds():
            pltpu.make_async_copy(
                rows.at[0], out_hbm.at[0, pl.ds(0, 128)], ssem).wait()

        fg(jnp.int32(0))
        fg(jnp.int32(1))

        def body(u, carry):
            dg()

            @pl.when(u >= 2)
            def _():
                ds()

            @pl.when(u + 2 < NUNITS)
            def _():
                fg(u + 2)

            fs(u)
            return carry

        lax.fori_loop(0, NUNITS, body, 0)
        ds()
        ds()

    return gather_kernel


_gather = _make_gather()


def kernel(indices, table):
    l = _gather(indices.astype(jnp.int32), table)
    return jnp.transpose(l, (1, 0, 2))


# speedup vs baseline: 3.5007x; 1.0715x over previous
import functools

import jax
import jax.numpy as jnp
import numpy as np
from jax import lax
from jax.experimental import pallas as pl
from jax.experimental.pallas import tpu as pltpu
from jax.experimental.pallas import tpu_sc as plsc

VOCAB = 100000
EMBED = 64
N_ROWS = 16384
N_COLS = 50

NUM_CORES = 2
NUM_SUBCORES = 16
NW = NUM_CORES * NUM_SUBCORES
R_PER_W = N_ROWS // NW
BT_PER_W = R_PER_W // 128
NUNITS = N_COLS * BT_PER_W
NSLOTS = 4
S_HALF = N_COLS // 2
NUNITS_H = S_HALF * BT_PER_W


def _make_gather(s_off):
    mesh = plsc.VectorSubcoreMesh(core_axis_name="c", subcore_axis_name="s")

    @functools.partial(
        pl.kernel,
        mesh=mesh,
        out_type=jax.ShapeDtypeStruct((S_HALF, N_ROWS, EMBED), jnp.float32),
        scratch_types=[
            pltpu.VMEM((R_PER_W, N_COLS), jnp.int32),
            pltpu.VMEM((S_HALF, BT_PER_W, 128), jnp.int32),
            pltpu.VMEM((NSLOTS, 128, EMBED), jnp.float32),
            pltpu.SemaphoreType.DMA,
            pltpu.SemaphoreType.DMA,
        ],
        compiler_params=pltpu.CompilerParams(
            use_tc_tiling_on_sc=False, needs_layout_passes=False),
    )
    def gather_kernel(idx_hbm, table_hbm, out_hbm, idx_v, idx_t, rows,
                      gsem, ssem):
        wid = lax.axis_index("s") * NUM_CORES + lax.axis_index("c")
        base_row = wid * R_PER_W
        pltpu.sync_copy(idx_hbm.at[pl.ds(base_row, R_PER_W)], idx_v)

        iota16 = lax.iota(jnp.int32, 16)

        def build_col(s, carry):
            sv = lax.broadcast(s + s_off, (16,))
            for bloc in range(BT_PER_W):
                for bc in range(8):
                    rowv = iota16 + (bloc * 128 + bc * 16)
                    v = plsc.load_gather(idx_v, [rowv, sv])
                    idx_t[s, bloc, pl.ds(bc * 16, 16)] = v
            return carry

        lax.fori_loop(0, S_HALF, build_col, 0)

        def fg(u):
            s = u // BT_PER_W
            bloc = lax.rem(u, BT_PER_W)
            slot = lax.rem(u, NSLOTS)
            pltpu.async_copy(
                table_hbm.at[idx_t.at[s, bloc]], rows.at[slot], gsem)

        def fs(u):
            s = u // BT_PER_W
            bloc = lax.rem(u, BT_PER_W)
            slot = lax.rem(u, NSLOTS)
            b0 = base_row + bloc * 128
            pltpu.async_copy(
                rows.at[slot], out_hbm.at[s, pl.ds(b0, 128)], ssem)

        def dg():
            pltpu.make_async_copy(
                table_hbm.at[pl.ds(0, 128)], rows.at[0], gsem).wait()

        def ds():
            pltpu.make_async_copy(
                rows.at[0], out_hbm.at[0, pl.ds(0, 128)], ssem).wait()

        fg(jnp.int32(0))
        fg(jnp.int32(1))

        def body(u, carry):
            dg()

            @pl.when(u >= 2)
            def _():
                ds()

            @pl.when(u + 2 < NUNITS_H)
            def _():
                fg(u + 2)

            fs(u)
            return carry

        lax.fori_loop(0, NUNITS_H, body, 0)
        ds()
        ds()

    return gather_kernel


_gather0 = _make_gather(0)
_gather1 = _make_gather(S_HALF)


def kernel(indices, table):
    l = _gather(indices.astype(jnp.int32), table)
    return jnp.transpose(l, (1, 0, 2))


# speedup vs baseline: 3.6757x; 1.0500x over previous
import functools

import jax
import jax.numpy as jnp
import numpy as np
from jax import lax
from jax.experimental import pallas as pl
from jax.experimental.pallas import tpu as pltpu
from jax.experimental.pallas import tpu_sc as plsc

VOCAB = 100000
EMBED = 64
N_ROWS = 16384
N_COLS = 50

NUM_CORES = 2
NUM_SUBCORES = 16
NW = NUM_CORES * NUM_SUBCORES
R_PER_W = N_ROWS // NW
BT_PER_W = R_PER_W // 128
NUNITS = N_COLS * BT_PER_W
NSLOTS = 4
S_HALF = N_COLS // 2
NUNITS_H = S_HALF * BT_PER_W


def _make_gather(s_off):
    mesh = plsc.VectorSubcoreMesh(core_axis_name="c", subcore_axis_name="s")

    @functools.partial(
        pl.kernel,
        mesh=mesh,
        out_type=jax.ShapeDtypeStruct((S_HALF, N_ROWS, EMBED), jnp.float32),
        scratch_types=[
            pltpu.VMEM((S_HALF, R_PER_W), jnp.int32),
            pltpu.VMEM((NSLOTS, 128, EMBED), jnp.float32),
            pltpu.SemaphoreType.DMA,
            pltpu.SemaphoreType.DMA,
        ],
        compiler_params=pltpu.CompilerParams(
            use_tc_tiling_on_sc=False, needs_layout_passes=False),
    )
    def gather_kernel(idx_hbm, table_hbm, out_hbm, idx_t, rows,
                      gsem, ssem):
        wid = lax.axis_index("s") * NUM_CORES + lax.axis_index("c")
        base_row = wid * R_PER_W
        pltpu.sync_copy(
            idx_hbm.at[pl.ds(s_off, S_HALF), pl.ds(base_row, R_PER_W)], idx_t)

        def fg(u):
            s = u // BT_PER_W
            bloc = lax.rem(u, BT_PER_W)
            slot = lax.rem(u, NSLOTS)
            pltpu.async_copy(
                table_hbm.at[idx_t.at[s, pl.ds(bloc * 128, 128)]],
                rows.at[slot], gsem)

        def fs(u):
            s = u // BT_PER_W
            bloc = lax.rem(u, BT_PER_W)
            slot = lax.rem(u, NSLOTS)
            b0 = base_row + bloc * 128
            pltpu.async_copy(
                rows.at[slot], out_hbm.at[s, pl.ds(b0, 128)], ssem)

        def dg():
            pltpu.make_async_copy(
                table_hbm.at[pl.ds(0, 128)], rows.at[0], gsem).wait()

        def ds():
            pltpu.make_async_copy(
                rows.at[0], out_hbm.at[0, pl.ds(0, 128)], ssem).wait()

        fg(jnp.int32(0))
        fg(jnp.int32(1))

        def body(u, carry):
            dg()

            @pl.when(u >= 2)
            def _():
                ds()

            @pl.when(u + 2 < NUNITS_H)
            def _():
                fg(u + 2)

            fs(u)
            return carry

        lax.fori_loop(0, NUNITS_H, body, 0)
        ds()
        ds()

    return gather_kernel


_gather0 = _make_gather(0)
_gather1 = _make_gather(S_HALF)


def kernel(indices, table):
    l = _gather(indices.astype(jnp.int32), table)
    return jnp.transpose(l, (1, 0, 2))


# speedup vs baseline: 3.7663x; 1.0246x over previous
import functools

import jax
import jax.numpy as jnp
import numpy as np
from jax import lax
from jax.experimental import pallas as pl
from jax.experimental.pallas import tpu as pltpu
from jax.experimental.pallas import tpu_sc as plsc

VOCAB = 100000
EMBED = 64
N_ROWS = 16384
N_COLS = 50

NUM_CORES = 2
NUM_SUBCORES = 16
NW = NUM_CORES * NUM_SUBCORES
R_PER_W = N_ROWS // NW
BT_PER_W = R_PER_W // 128
NUNITS = N_COLS * BT_PER_W
NSLOTS = 4
S_HALF = 10
NUNITS_H = S_HALF * BT_PER_W


def _make_gather(s_off):
    mesh = plsc.VectorSubcoreMesh(core_axis_name="c", subcore_axis_name="s")

    @functools.partial(
        pl.kernel,
        mesh=mesh,
        out_type=jax.ShapeDtypeStruct((S_HALF, N_ROWS, EMBED), jnp.float32),
        scratch_types=[
            pltpu.VMEM((S_HALF, R_PER_W), jnp.int32),
            pltpu.VMEM((NSLOTS, 128, EMBED), jnp.float32),
            pltpu.SemaphoreType.DMA,
            pltpu.SemaphoreType.DMA,
        ],
        compiler_params=pltpu.CompilerParams(
            use_tc_tiling_on_sc=False, needs_layout_passes=False),
    )
    def gather_kernel(idx_hbm, table_hbm, out_hbm, idx_t, rows,
                      gsem, ssem):
        wid = lax.axis_index("s") * NUM_CORES + lax.axis_index("c")
        base_row = wid * R_PER_W
        pltpu.sync_copy(
            idx_hbm.at[pl.ds(s_off, S_HALF), pl.ds(base_row, R_PER_W)], idx_t)

        def fg(u):
            s = u // BT_PER_W
            bloc = lax.rem(u, BT_PER_W)
            slot = lax.rem(u, NSLOTS)
            pltpu.async_copy(
                table_hbm.at[idx_t.at[s, pl.ds(bloc * 128, 128)]],
                rows.at[slot], gsem)

        def fs(u):
            s = u // BT_PER_W
            bloc = lax.rem(u, BT_PER_W)
            slot = lax.rem(u, NSLOTS)
            b0 = base_row + bloc * 128
            pltpu.async_copy(
                rows.at[slot], out_hbm.at[s, pl.ds(b0, 128)], ssem)

        def dg():
            pltpu.make_async_copy(
                table_hbm.at[pl.ds(0, 128)], rows.at[0], gsem).wait()

        def ds():
            pltpu.make_async_copy(
                rows.at[0], out_hbm.at[0, pl.ds(0, 128)], ssem).wait()

        fg(jnp.int32(0))
        fg(jnp.int32(1))

        def body(u, carry):
            dg()

            @pl.when(u >= 2)
            def _():
                ds()

            @pl.when(u + 2 < NUNITS_H)
            def _():
                fg(u + 2)

            fs(u)
            return carry

        lax.fori_loop(0, NUNITS_H, body, 0)
        ds()
        ds()

    return gather_kernel


_gathers = [_make_gather(i * S_HALF) for i in range(N_COLS // S_HALF)]


def kernel(indices, table):
    l = _gather(indices.astype(jnp.int32), table)
    return jnp.transpose(l, (1, 0, 2))


# speedup vs baseline: 3.7685x; 1.0006x over previous
import functools

import jax
import jax.numpy as jnp
import numpy as np
from jax import lax
from jax.experimental import pallas as pl
from jax.experimental.pallas import tpu as pltpu
from jax.experimental.pallas import tpu_sc as plsc

VOCAB = 100000
EMBED = 64
N_ROWS = 16384
N_COLS = 50

NUM_CORES = 2
NUM_SUBCORES = 16
NW = NUM_CORES * NUM_SUBCORES
R_PER_W = N_ROWS // NW
BT_PER_W = R_PER_W // 128
NUNITS = N_COLS * BT_PER_W
NSLOTS = 4
S_HALF = 10
NUNITS_H = S_HALF * BT_PER_W


def _make_gather(s_off):
    mesh = plsc.VectorSubcoreMesh(core_axis_name="c", subcore_axis_name="s")

    @functools.partial(
        pl.kernel,
        mesh=mesh,
        out_type=jax.ShapeDtypeStruct((S_HALF, N_ROWS, EMBED), jnp.float32),
        scratch_types=[
            pltpu.VMEM((S_HALF, R_PER_W), jnp.int32),
            pltpu.VMEM((NSLOTS, 128, EMBED), jnp.float32),
            pltpu.SemaphoreType.DMA,
            pltpu.SemaphoreType.DMA,
        ],
        compiler_params=pltpu.CompilerParams(
            use_tc_tiling_on_sc=False, needs_layout_passes=False),
    )
    def gather_kernel(idx_hbm, table_hbm, out_hbm, idx_t, rows,
                      gsem, ssem):
        wid = lax.axis_index("s") * NUM_CORES + lax.axis_index("c")
        base_row = wid * R_PER_W
        pltpu.sync_copy(
            idx_hbm.at[pl.ds(s_off, S_HALF), pl.ds(base_row, R_PER_W)], idx_t)

        def fg(u):
            s = u // BT_PER_W
            bloc = lax.rem(u, BT_PER_W)
            slot = lax.rem(u, NSLOTS)
            pltpu.async_copy(
                table_hbm.at[idx_t.at[s, pl.ds(bloc * 128, 128)]],
                rows.at[slot], gsem)

        def fs(u):
            s = u // BT_PER_W
            bloc = lax.rem(u, BT_PER_W)
            slot = lax.rem(u, NSLOTS)
            b0 = base_row + bloc * 128
            pltpu.async_copy(
                rows.at[slot], out_hbm.at[s, pl.ds(b0, 128)], ssem)

        def dg():
            pltpu.make_async_copy(
                table_hbm.at[pl.ds(0, 128)], rows.at[0], gsem).wait()

        def ds():
            pltpu.make_async_copy(
                rows.at[0], out_hbm.at[0, pl.ds(0, 128)], ssem).wait()

        fg(jnp.int32(0))
        fg(jnp.int32(1))

        def body(u, carry):
            dg()

            @pl.when(u >= 2)
            def _():
                ds()

            @pl.when(u + 2 < NUNITS_H)
            def _():
                fg(u + 2)

            fs(u)
            return carry

        lax.fori_loop(0, NUNITS_H, body, 0)
        ds()
        ds()

    return gather_kernel


_gathers = [_make_gather(i * S_HALF) for i in range(N_COLS // S_HALF)]

_P0_NP = np.zeros((64, 128), np.float32)
_P0_NP[np.arange(64), 2 * np.arange(64)] = 1.0
_P1_NP = np.zeros((64, 128), np.float32)
_P1_NP[np.arange(64), 2 * np.arange(64) + 1] = 1.0


def _tc_core(x_ref, p0_ref, p1_ref, out_ref):
    x = x_ref[...]
    p0 = p0_ref[...]
    p1 = p1_ref[...]
    dn = (((0,), (0,)), ((), ()))
    for btl in range(128):
        xs = x[btl * 64:(btl + 1) * 64, :]
        t = (lax.dot_general(xs[:, 0:64], p0, dn,
                             preferred_element_type=jnp.float32)
             + lax.dot_general(xs[:, 64:128], p1, dn,
                               preferred_element_type=jnp.float32))
        out_ref[0, :, btl, :, :] = t.reshape(8, 8, 128)


def _tc_body1(x_ref, p0_ref, p1_ref, pin_ref, out_ref):
    del pin_ref
    _tc_core(x_ref, p0_ref, p1_ref, out_ref)


_P_SHAPE = jax.ShapeDtypeStruct((N_COLS, 8, 128, 8, 128), jnp.float32)

def _make_tc(chunk):
    off = chunk * S_HALF
    in_specs = [
        pl.BlockSpec((8192, 128), lambda s: (s, 0)),
        pl.BlockSpec((64, 128), lambda s: (0, 0)),
        pl.BlockSpec((64, 128), lambda s: (0, 0)),
    ]
    if chunk == 0:
        return pl.pallas_call(
            _tc_core,
            grid=(S_HALF,),
            in_specs=in_specs,
            out_specs=pl.BlockSpec((1, 8, 128, 8, 128),
                                   lambda s: (s, 0, 0, 0, 0)),
            out_shape=_P_SHAPE,
        )
    return pl.pallas_call(
        _tc_body1,
        grid=(S_HALF,),
        in_specs=in_specs + [pl.BlockSpec(memory_space=pl.ANY)],
        out_specs=pl.BlockSpec((1, 8, 128, 8, 128),
                               lambda s, _o=off: (s + _o, 0, 0, 0, 0)),
        out_shape=_P_SHAPE,
        input_output_aliases={3: 0},
    )


_tcs = [_make_tc(i) for i in range(N_COLS // S_HALF)]

_X_ROWS = S_HALF * N_ROWS * EMBED // 128


def kernel(indices, table):
    idx32 = jnp.transpose(indices).astype(jnp.int32)
    p0m = jnp.asarray(_P0_NP)
    p1m = jnp.asarray(_P1_NP)
    ls = [g(idx32, table) for g in _gathers]
    p = _tcs[0](ls[0].reshape(_X_ROWS, 128), p0m, p1m)
    for i in range(1, len(_tcs)):
        p = _tcs[i](ls[i].reshape(_X_ROWS, 128), p0m, p1m, p)
    return jnp.transpose(p, (2, 4, 0, 1, 3)).reshape(N_ROWS, N_COLS, EMBED)
